# baseline (speedup 1.0000x reference)
import jax
import jax.numpy as jnp
from jax import lax
from jax.experimental import pallas as pl
from jax.experimental.pallas import tpu as pltpu

BM = 200


def _dot(a, b):
    return jnp.dot(a, b, preferred_element_type=jnp.float32)


def _dot_t(a, w):
    return lax.dot_general(a, w, (((1,), (1,)), ((), ())),
                           preferred_element_type=jnp.float32)


def _make_sfgcn_kernel(nph, bm):
    def _sfgcn_kernel(sadj_ref, fadj_ref, x_ref,
                      s1w1_ref, s1b1_ref, s1w2_ref, s1b2_ref,
                      s2w1_ref, s2b1_ref, s2w2_ref, s2b2_ref,
                      cw1_ref, cb1_ref, cw2_ref, cb2_ref,
                      aw1_ref, ab1_ref, aw2_ref, mw_ref, mb_ref,
                      out_ref,
                      sup1_s, sup1_f, sup2_s, sup2_f):
        i = pl.program_id(0)
        nh = s1w2_ref.shape[0]

        @pl.when(i == 0)
        def _prologue():
            xv = x_ref[...]
            com = _dot_t(xv, cw1_ref[...]) + cb1_ref[...]
            sup1_s[:, :nh] = _dot_t(xv, s1w1_ref[...]) + s1b1_ref[...]
            sup1_s[:, nh:] = com
            sup1_f[:, :nh] = _dot_t(xv, s2w1_ref[...]) + s2b1_ref[...]
            sup1_f[:, nh:] = com

        @pl.when(i < nph)
        def _layer1():
            rows = pl.ds(pl.multiple_of(i * bm, 8), bm)
            h_s = jax.nn.relu(_dot(sadj_ref[...].astype(jnp.bfloat16),
                                   sup1_s[...].astype(jnp.bfloat16)))
            h_f = jax.nn.relu(_dot(fadj_ref[...].astype(jnp.bfloat16),
                                   sup1_f[...].astype(jnp.bfloat16)))
            sup2_s[rows, :nh] = _dot_t(h_s[:, :nh], s1w2_ref[...]) + s1b2_ref[...]
            sup2_s[rows, nh:] = _dot_t(h_s[:, nh:], cw2_ref[...]) + cb2_ref[...]
            sup2_f[rows, :nh] = _dot_t(h_f[:, :nh], s2w2_ref[...]) + s2b2_ref[...]
            sup2_f[rows, nh:] = _dot_t(h_f[:, nh:], cw2_ref[...]) + cb2_ref[...]

        @pl.when(i >= nph)
        def _layer2():
            out_s = _dot(sadj_ref[...].astype(jnp.bfloat16),
                         sup2_s[...].astype(jnp.bfloat16))
            out_f = _dot(fadj_ref[...].astype(jnp.bfloat16),
                         sup2_f[...].astype(jnp.bfloat16))
            emb1 = out_s[:, :nh]
            com1 = out_s[:, nh:]
            emb2 = out_f[:, :nh]
            com2 = out_f[:, nh:]
            xcom = 0.5 * (com1 + com2)

            aw1 = aw1_ref[...]
            ab1 = ab1_ref[...]
            aw2 = aw2_ref[...]

            def att_logit(e):
                a = jnp.tanh(_dot_t(e, aw1) + ab1)
                return jnp.sum(a * aw2, axis=1, keepdims=True)

            w1 = att_logit(emb1)
            w2 = att_logit(emb2)
            w3 = att_logit(xcom)
            m = jnp.maximum(jnp.maximum(w1, w2), w3)
            e1 = jnp.exp(w1 - m)
            e2 = jnp.exp(w2 - m)
            e3 = jnp.exp(w3 - m)
            emb_att = (e1 * emb1 + e2 * emb2 + e3 * xcom) / (e1 + e2 + e3)
            out_ref[...] = _dot_t(emb_att, mw_ref[...]) + mb_ref[...]

    return _sfgcn_kernel


def kernel(x, sadj, fadj,
           s1_w1, s1_b1, s1_w2, s1_b2,
           s2_w1, s2_b1, s2_w2, s2_b2,
           c_w1, c_b1, c_w2, c_b2,
           att_w1, att_b1, att_w2,
           mlp_w, mlp_b):
    n, nfeat = x.shape
    nhid = s1_w2.shape[0]
    nclass = mlp_w.shape[0]
    w = 2 * nhid

    row = lambda v: v.reshape(1, -1)

    blk = BM if n % BM == 0 else n
    nph = n // blk
    adjspec = pl.BlockSpec((blk, n), lambda i: (jnp.where(i < nph, i, i - nph), 0))
    full = lambda shape: pl.BlockSpec(shape, lambda i: (0, 0))

    out = pl.pallas_call(
        _make_sfgcn_kernel(nph, blk),
        grid=(2 * nph,),
        in_specs=[adjspec, adjspec,
                  full((n, nfeat)),
                  full((nhid, nfeat)), full((1, nhid)),
                  full((nhid, nhid)), full((1, nhid)),
                  full((nhid, nfeat)), full((1, nhid)),
                  full((nhid, nhid)), full((1, nhid)),
                  full((nhid, nfeat)), full((1, nhid)),
                  full((nhid, nhid)), full((1, nhid)),
                  full((16, nhid)), full((1, 16)), full((1, 16)),
                  full((nclass, nhid)), full((1, nclass))],
        out_specs=pl.BlockSpec((blk, nclass),
                               lambda i: (jnp.where(i < nph, 0, i - nph), 0)),
        out_shape=jax.ShapeDtypeStruct((n, nclass), jnp.float32),
        scratch_shapes=[pltpu.VMEM((n, w), jnp.float32)] * 4,
        compiler_params=pltpu.CompilerParams(dimension_semantics=("arbitrary",),
                                             vmem_limit_bytes=63 * 1024 * 1024),
    )(sadj, fadj, x,
      s1_w1, row(s1_b1), s1_w2, row(s1_b2),
      s2_w1, row(s2_b1), s2_w2, row(s2_b2),
      c_w1, row(c_b1), c_w2, row(c_b2),
      att_w1, row(att_b1), row(att_w2), mlp_w, row(mlp_b))

    return out


# speedup vs baseline: 1.0044x; 1.0044x over previous
import jax
import jax.numpy as jnp
from jax import lax
from jax.experimental import pallas as pl
from jax.experimental.pallas import tpu as pltpu

BM = 200


def _dot(a, b):
    return jnp.dot(a, b, preferred_element_type=jnp.float32)


def _dot_t(a, w):
    return lax.dot_general(a, w, (((1,), (1,)), ((), ())),
                           preferred_element_type=jnp.float32)


def _make_sfgcn_kernel(nph, bm):
    def _sfgcn_kernel(sadj_ref, fadj_ref, x_ref,
                      s1w1_ref, s1b1_ref, s1w2_ref, s1b2_ref,
                      s2w1_ref, s2b1_ref, s2w2_ref, s2b2_ref,
                      cw1_ref, cb1_ref, cw2_ref, cb2_ref,
                      aw1_ref, ab1_ref, aw2_ref, mw_ref, mb_ref,
                      out_ref,
                      sup1_s, sup1_f, sup2_s, sup2_f):
        i = pl.program_id(0)
        nh = s1w2_ref.shape[0]

        @pl.when(i == 0)
        def _prologue():
            xv = x_ref[...]
            com = _dot_t(xv, cw1_ref[...]) + cb1_ref[...]
            sup1_s[:, :nh] = _dot_t(xv, s1w1_ref[...]) + s1b1_ref[...]
            sup1_s[:, nh:] = com
            sup1_f[:, :nh] = _dot_t(xv, s2w1_ref[...]) + s2b1_ref[...]
            sup1_f[:, nh:] = com

        @pl.when(i < nph)
        def _layer1():
            rows = pl.ds(pl.multiple_of(i * bm, 8), bm)
            h_s = jax.nn.relu(_dot(sadj_ref[...], sup1_s[...]))
            h_f = jax.nn.relu(_dot(fadj_ref[...], sup1_f[...]))
            sup2_s[rows, :nh] = _dot_t(h_s[:, :nh], s1w2_ref[...]) + s1b2_ref[...]
            sup2_s[rows, nh:] = _dot_t(h_s[:, nh:], cw2_ref[...]) + cb2_ref[...]
            sup2_f[rows, :nh] = _dot_t(h_f[:, :nh], s2w2_ref[...]) + s2b2_ref[...]
            sup2_f[rows, nh:] = _dot_t(h_f[:, nh:], cw2_ref[...]) + cb2_ref[...]

        @pl.when(i >= nph)
        def _layer2():
            out_s = _dot(sadj_ref[...], sup2_s[...])
            out_f = _dot(fadj_ref[...], sup2_f[...])
            emb1 = out_s[:, :nh]
            com1 = out_s[:, nh:]
            emb2 = out_f[:, :nh]
            com2 = out_f[:, nh:]
            xcom = 0.5 * (com1 + com2)

            aw1 = aw1_ref[...]
            ab1 = ab1_ref[...]
            aw2 = aw2_ref[...]

            def att_logit(e):
                a = jnp.tanh(_dot_t(e, aw1) + ab1)
                return jnp.sum(a * aw2, axis=1, keepdims=True)

            w1 = att_logit(emb1)
            w2 = att_logit(emb2)
            w3 = att_logit(xcom)
            m = jnp.maximum(jnp.maximum(w1, w2), w3)
            e1 = jnp.exp(w1 - m)
            e2 = jnp.exp(w2 - m)
            e3 = jnp.exp(w3 - m)
            emb_att = (e1 * emb1 + e2 * emb2 + e3 * xcom) / (e1 + e2 + e3)
            out_ref[...] = _dot_t(emb_att, mw_ref[...]) + mb_ref[...]

    return _sfgcn_kernel


def kernel(x, sadj, fadj,
           s1_w1, s1_b1, s1_w2, s1_b2,
           s2_w1, s2_b1, s2_w2, s2_b2,
           c_w1, c_b1, c_w2, c_b2,
           att_w1, att_b1, att_w2,
           mlp_w, mlp_b):
    n, nfeat = x.shape
    nhid = s1_w2.shape[0]
    nclass = mlp_w.shape[0]
    w = 2 * nhid

    row = lambda v: v.reshape(1, -1)

    blk = BM if n % BM == 0 else n
    nph = n // blk
    adjspec = pl.BlockSpec((blk, n), lambda i: (jnp.where(i < nph, i, i - nph), 0))
    full = lambda shape: pl.BlockSpec(shape, lambda i: (0, 0))

    out = pl.pallas_call(
        _make_sfgcn_kernel(nph, blk),
        grid=(2 * nph,),
        in_specs=[adjspec, adjspec,
                  full((n, nfeat)),
                  full((nhid, nfeat)), full((1, nhid)),
                  full((nhid, nhid)), full((1, nhid)),
                  full((nhid, nfeat)), full((1, nhid)),
                  full((nhid, nhid)), full((1, nhid)),
                  full((nhid, nfeat)), full((1, nhid)),
                  full((nhid, nhid)), full((1, nhid)),
                  full((16, nhid)), full((1, 16)), full((1, 16)),
                  full((nclass, nhid)), full((1, nclass))],
        out_specs=pl.BlockSpec((blk, nclass),
                               lambda i: (jnp.where(i < nph, 0, i - nph), 0)),
        out_shape=jax.ShapeDtypeStruct((n, nclass), jnp.float32),
        scratch_shapes=[pltpu.VMEM((n, w), jnp.float32)] * 4,
        compiler_params=pltpu.CompilerParams(dimension_semantics=("arbitrary",),
                                             vmem_limit_bytes=63 * 1024 * 1024),
    )(sadj, fadj, x,
      s1_w1, row(s1_b1), s1_w2, row(s1_b2),
      s2_w1, row(s2_b1), s2_w2, row(s2_b2),
      c_w1, row(c_b1), c_w2, row(c_b2),
      att_w1, row(att_b1), row(att_w2), mlp_w, row(mlp_b))

    return out
